# baseline (speedup 1.0000x reference)
import functools

import jax
import jax.numpy as jnp
from jax import lax
from jax.experimental import pallas as pl
from jax.experimental.pallas import tpu as pltpu
from jax.experimental.pallas import tpu_sc as plsc

B = 512
D = 512
R = 16384
NRG = 3
SPG = 32
S = 100000
G = R * NRG
NCAND = NRG * SPG
EPS = 1e-9

R_BLK = 4096
NEG_INF = float('-inf')

NWORKER = 32
QPW = B // NWORKER


def _reaction_body(q_ref, k_ref, gr_ref, react_ref, sel_ref,
                   max_sc, idx_sc, sel_sc):
    i = pl.program_id(0)

    @pl.when(i == 0)
    def _init():
        max_sc[...] = jnp.full((B,), NEG_INF, jnp.float32)
        idx_sc[...] = jnp.zeros((B,), jnp.int32)
        sel_sc[...] = jnp.zeros((B,), jnp.float32)

    logits = lax.dot_general(
        q_ref[...], k_ref[...],
        dimension_numbers=(((1,), (1,)), ((), ())),
        preferred_element_type=jnp.float32)
    g_r = -jnp.log(-jnp.log(gr_ref[...] + EPS) + EPS)
    pert = logits + g_r
    pmax = jnp.max(pert, axis=1)
    iota = lax.broadcasted_iota(jnp.int32, (B, R_BLK), 1)
    loc = jnp.min(jnp.where(pert == pmax[:, None], iota, R_BLK), axis=1)
    sel = jnp.max(jnp.where(iota == loc[:, None], logits, NEG_INF), axis=1)

    better = pmax > max_sc[...]
    max_sc[...] = jnp.where(better, pmax, max_sc[...])
    idx_sc[...] = jnp.where(better, loc + i * R_BLK, idx_sc[...])
    sel_sc[...] = jnp.where(better, sel, sel_sc[...])

    @pl.when(i == pl.num_programs(0) - 1)
    def _done():
        react_ref[...] = idx_sc[...]
        sel_ref[...] = sel_sc[...]


def _reaction_phase(queries, reaction_keys, g_r):
    return pl.pallas_call(
        _reaction_body,
        grid=(R // R_BLK,),
        in_specs=[
            pl.BlockSpec((B, D), lambda i: (0, 0)),
            pl.BlockSpec((R_BLK, D), lambda i: (i, 0)),
            pl.BlockSpec((B, R_BLK), lambda i: (0, i)),
        ],
        out_specs=[
            pl.BlockSpec((B,), lambda i: (0,)),
            pl.BlockSpec((B,), lambda i: (0,)),
        ],
        out_shape=[
            jax.ShapeDtypeStruct((B,), jnp.int32),
            jax.ShapeDtypeStruct((B,), jnp.float32),
        ],
        scratch_shapes=[
            pltpu.VMEM((B,), jnp.float32),
            pltpu.VMEM((B,), jnp.int32),
            pltpu.VMEM((B,), jnp.float32),
        ],
        compiler_params=pltpu.CompilerParams(
            dimension_semantics=("arbitrary",)),
    )(queries, reaction_keys, g_r)


def _rne_bf16(v):
    x = plsc.bitcast(v, jnp.int32)
    lsb = (x >> 16) & 1
    r = (x + (0x7FFF + lsb)) & jnp.int32(-65536)
    return plsc.bitcast(r, jnp.float32)


def _synthon_sc_body(react_hbm, q_hbm, r2s_hbm, skeys_hbm,
                     out_logits_hbm, out_ids_hbm,
                     react_v, flat_v, q_v, rows2,
                     part_t, stage_v, sem0, sem1):
    wid = lax.axis_index("s") * 2 + lax.axis_index("c")
    base = wid * QPW

    pltpu.sync_copy(react_hbm.at[pl.ds(base, QPW)], react_v)
    pltpu.sync_copy(q_hbm.at[pl.ds(base, QPW)], q_v)

    iota = lax.iota(jnp.int32, 16)
    r_vec = react_v[...]
    for b in range(QPW):
        rc = r_vec[b]
        pltpu.sync_copy(r2s_hbm.at[pl.ds(rc, 1)], flat_v.at[pl.ds(b, 1)])
    pltpu.sync_copy(flat_v, out_ids_hbm.at[pl.ds(base, QPW)])

    pltpu.async_copy(skeys_hbm.at[flat_v.at[0]], rows2.at[0], sem0)

    for b in range(QPW):
        p = b % 2
        cur_sem = sem0 if p == 0 else sem1
        if b < QPW - 1:
            nxt_sem = sem1 if p == 0 else sem0
            pltpu.async_copy(skeys_hbm.at[flat_v.at[b + 1]],
                             rows2.at[1 - p], nxt_sem)
        pltpu.make_async_copy(skeys_hbm.at[flat_v.at[b]],
                              rows2.at[p], cur_sem).wait()

        for c in range(D // 16):
            q_v[b, pl.ds(c * 16, 16)] = _rne_bf16(q_v[b, pl.ds(c * 16, 16)])

        def _cand_body(ci, _c, p=p, b=b):
            accs = [jnp.zeros((16,), jnp.float32) for _ in range(4)]
            for c in range(D // 16):
                k = _rne_bf16(rows2[p, ci, pl.ds(c * 16, 16)])
                accs[c % 4] = accs[c % 4] + k * q_v[b, pl.ds(c * 16, 16)]
            acc = (accs[0] + accs[1]) + (accs[2] + accs[3])
            plsc.store_scatter(part_t, [iota, jnp.full((16,), ci, jnp.int32)],
                               acc)
            return _c
        lax.fori_loop(0, NCAND, _cand_body, 0)

        def _red_body(g, _r2, b=b):
            tot = jnp.zeros((16,), jnp.float32)
            for k in range(16):
                tot = tot + part_t[k, pl.ds(g * 16, 16)]
            stage_v[b, pl.ds(g * 16, 16)] = tot
            return _r2
        lax.fori_loop(0, NCAND // 16, _red_body, 0)

    pltpu.sync_copy(stage_v, out_logits_hbm.at[pl.ds(base, QPW)])


def _synthon_phase(reactions, queries_rnd, rgroup2synthons, synthon_keys):
    mesh = plsc.VectorSubcoreMesh(core_axis_name="c", subcore_axis_name="s")
    fn = pl.kernel(
        _synthon_sc_body,
        out_type=[
            jax.ShapeDtypeStruct((B, NCAND), jnp.float32),
            jax.ShapeDtypeStruct((B, NCAND), jnp.int32),
        ],
        mesh=mesh,
        scratch_types=[
            pltpu.VMEM((QPW,), jnp.int32),
            pltpu.VMEM((QPW, NCAND), jnp.int32),
            pltpu.VMEM((QPW, D), jnp.float32),
            pltpu.VMEM((2, NCAND, D), jnp.float32),
            pltpu.VMEM((16, NCAND), jnp.float32),
            pltpu.VMEM((QPW, NCAND), jnp.float32),
            pltpu.SemaphoreType.DMA,
            pltpu.SemaphoreType.DMA,
        ],
        compiler_params=pltpu.CompilerParams(needs_layout_passes=False),
    )
    return fn(reactions, queries_rnd, rgroup2synthons, synthon_keys)


def _epilogue_body(lg_ref, gs_ref, ids_ref, sel_ref, syn_ref, sc_ref):
    pert = lg_ref[...] + gs_ref[...]
    ids = ids_ref[...]
    cols = []
    maxes = [sel_ref[...][:, None]]
    io = lax.broadcasted_iota(jnp.int32, (B, SPG), 1)
    for j in range(NRG):
        blk = pert[:, j * SPG:(j + 1) * SPG]
        idb = ids[:, j * SPG:(j + 1) * SPG]
        m = jnp.max(blk, axis=1)
        ch = jnp.min(jnp.where(blk == m[:, None], io, SPG), axis=1)
        sid = jnp.max(jnp.where(io == ch[:, None], idb, -1), axis=1)
        cols.append(sid[:, None])
        maxes.append(m[:, None])
    syn_ref[...] = jnp.concatenate(cols, axis=1)
    sc_ref[...] = jnp.concatenate(maxes, axis=1)


def _epilogue(syn_logits, g_s2, syn_ids, sel_logit):
    return pl.pallas_call(
        _epilogue_body,
        out_shape=[
            jax.ShapeDtypeStruct((B, NRG), jnp.int32),
            jax.ShapeDtypeStruct((B, 1 + NRG), jnp.float32),
        ],
    )(syn_logits, g_s2, syn_ids, sel_logit)


def kernel(queries, reaction_keys, synthon_keys, rgroup2synthons, noise_r, noise_s):
    g_s = -jnp.log(-jnp.log(noise_s + EPS) + EPS)

    reactions, sel_logit = _reaction_phase(queries, reaction_keys, noise_r)

    syn_logits, syn_ids = _synthon_phase(
        reactions, queries,
        rgroup2synthons.reshape(R, NRG * SPG), synthon_keys)

    synthons, scores = _epilogue(
        syn_logits, g_s.reshape(B, NCAND), syn_ids, sel_logit)
    return reactions, synthons, scores


# speedup vs baseline: 1.0557x; 1.0557x over previous
import functools

import jax
import jax.numpy as jnp
from jax import lax
from jax.experimental import pallas as pl
from jax.experimental.pallas import tpu as pltpu
from jax.experimental.pallas import tpu_sc as plsc

B = 512
D = 512
R = 16384
NRG = 3
SPG = 32
S = 100000
G = R * NRG
NCAND = NRG * SPG
EPS = 1e-9

R_BLK = 2048
NEG_INF = float('-inf')

NWORKER = 32
QPW = B // NWORKER


def _reaction_body(q_ref, k_ref, gr_ref, react_ref, sel_ref,
                   max_sc, idx_sc, sel_sc):
    i = pl.program_id(0)

    @pl.when(i == 0)
    def _init():
        max_sc[...] = jnp.full((B,), NEG_INF, jnp.float32)
        idx_sc[...] = jnp.zeros((B,), jnp.int32)
        sel_sc[...] = jnp.zeros((B,), jnp.float32)

    logits = lax.dot_general(
        q_ref[...], k_ref[...],
        dimension_numbers=(((1,), (1,)), ((), ())),
        preferred_element_type=jnp.float32)
    g_r = -jnp.log(-jnp.log(gr_ref[...] + EPS) + EPS)
    pert = logits + g_r
    pmax = jnp.max(pert, axis=1)
    iota = lax.broadcasted_iota(jnp.int32, (B, R_BLK), 1)
    loc = jnp.min(jnp.where(pert == pmax[:, None], iota, R_BLK), axis=1)
    sel = jnp.max(jnp.where(iota == loc[:, None], logits, NEG_INF), axis=1)

    better = pmax > max_sc[...]
    max_sc[...] = jnp.where(better, pmax, max_sc[...])
    idx_sc[...] = jnp.where(better, loc + i * R_BLK, idx_sc[...])
    sel_sc[...] = jnp.where(better, sel, sel_sc[...])

    @pl.when(i == pl.num_programs(0) - 1)
    def _done():
        react_ref[...] = idx_sc[...]
        sel_ref[...] = sel_sc[...]


def _reaction_phase(queries, reaction_keys, g_r):
    return pl.pallas_call(
        _reaction_body,
        grid=(R // R_BLK,),
        in_specs=[
            pl.BlockSpec((B, D), lambda i: (0, 0)),
            pl.BlockSpec((R_BLK, D), lambda i: (i, 0)),
            pl.BlockSpec((B, R_BLK), lambda i: (0, i)),
        ],
        out_specs=[
            pl.BlockSpec((B,), lambda i: (0,)),
            pl.BlockSpec((B,), lambda i: (0,)),
        ],
        out_shape=[
            jax.ShapeDtypeStruct((B,), jnp.int32),
            jax.ShapeDtypeStruct((B,), jnp.float32),
        ],
        scratch_shapes=[
            pltpu.VMEM((B,), jnp.float32),
            pltpu.VMEM((B,), jnp.int32),
            pltpu.VMEM((B,), jnp.float32),
        ],
        compiler_params=pltpu.CompilerParams(
            dimension_semantics=("arbitrary",)),
    )(queries, reaction_keys, g_r)


def _rne_bf16(v):
    x = plsc.bitcast(v, jnp.int32)
    lsb = (x >> 16) & 1
    r = (x + (0x7FFF + lsb)) & jnp.int32(-65536)
    return plsc.bitcast(r, jnp.float32)


def _synthon_sc_body(react_hbm, q_hbm, r2s_hbm, skeys_hbm,
                     out_logits_hbm, out_ids_hbm,
                     react_v, flat_v, q_v, rows2,
                     part_t, stage_v, sem0, sem1):
    wid = lax.axis_index("s") * 2 + lax.axis_index("c")
    base = wid * QPW

    pltpu.sync_copy(react_hbm.at[pl.ds(base, QPW)], react_v)
    pltpu.sync_copy(q_hbm.at[pl.ds(base, QPW)], q_v)

    iota = lax.iota(jnp.int32, 16)
    r_vec = react_v[...]
    for b in range(QPW):
        rc = r_vec[b]
        pltpu.async_copy(r2s_hbm.at[pl.ds(rc, 1)], flat_v.at[pl.ds(b, 1)],
                         sem1)
    for b in range(QPW):
        pltpu.make_async_copy(r2s_hbm.at[pl.ds(0, 1)],
                              flat_v.at[pl.ds(b, 1)], sem1).wait()
    pltpu.sync_copy(flat_v, out_ids_hbm.at[pl.ds(base, QPW)])

    pltpu.async_copy(skeys_hbm.at[flat_v.at[0]], rows2.at[0], sem0)

    for b in range(QPW):
        p = b % 2
        cur_sem = sem0 if p == 0 else sem1
        if b < QPW - 1:
            nxt_sem = sem1 if p == 0 else sem0
            pltpu.async_copy(skeys_hbm.at[flat_v.at[b + 1]],
                             rows2.at[1 - p], nxt_sem)
        pltpu.make_async_copy(skeys_hbm.at[flat_v.at[b]],
                              rows2.at[p], cur_sem).wait()

        for c in range(D // 16):
            q_v[b, pl.ds(c * 16, 16)] = _rne_bf16(q_v[b, pl.ds(c * 16, 16)])

        def _cand_body(ci, _c, p=p, b=b):
            accs = [jnp.zeros((16,), jnp.float32) for _ in range(4)]
            for c in range(D // 16):
                k = _rne_bf16(rows2[p, ci, pl.ds(c * 16, 16)])
                accs[c % 4] = accs[c % 4] + k * q_v[b, pl.ds(c * 16, 16)]
            acc = (accs[0] + accs[1]) + (accs[2] + accs[3])
            plsc.store_scatter(part_t, [iota, jnp.full((16,), ci, jnp.int32)],
                               acc)
            return _c
        lax.fori_loop(0, NCAND, _cand_body, 0)

        def _red_body(g, _r2, b=b):
            tot = jnp.zeros((16,), jnp.float32)
            for k in range(16):
                tot = tot + part_t[k, pl.ds(g * 16, 16)]
            stage_v[b, pl.ds(g * 16, 16)] = tot
            return _r2
        lax.fori_loop(0, NCAND // 16, _red_body, 0)

    pltpu.sync_copy(stage_v, out_logits_hbm.at[pl.ds(base, QPW)])


def _synthon_phase(reactions, queries_rnd, rgroup2synthons, synthon_keys):
    mesh = plsc.VectorSubcoreMesh(core_axis_name="c", subcore_axis_name="s")
    fn = pl.kernel(
        _synthon_sc_body,
        out_type=[
            jax.ShapeDtypeStruct((B, NCAND), jnp.float32),
            jax.ShapeDtypeStruct((B, NCAND), jnp.int32),
        ],
        mesh=mesh,
        scratch_types=[
            pltpu.VMEM((QPW,), jnp.int32),
            pltpu.VMEM((QPW, NCAND), jnp.int32),
            pltpu.VMEM((QPW, D), jnp.float32),
            pltpu.VMEM((2, NCAND, D), jnp.float32),
            pltpu.VMEM((16, NCAND), jnp.float32),
            pltpu.VMEM((QPW, NCAND), jnp.float32),
            pltpu.SemaphoreType.DMA,
            pltpu.SemaphoreType.DMA,
        ],
        compiler_params=pltpu.CompilerParams(needs_layout_passes=False),
    )
    return fn(reactions, queries_rnd, rgroup2synthons, synthon_keys)


def _epilogue_body(lg_ref, gs_ref, ids_ref, sel_ref, syn_ref, sc_ref):
    pert = lg_ref[...] + gs_ref[...]
    ids = ids_ref[...]
    cols = []
    maxes = [sel_ref[...][:, None]]
    io = lax.broadcasted_iota(jnp.int32, (B, SPG), 1)
    for j in range(NRG):
        blk = pert[:, j * SPG:(j + 1) * SPG]
        idb = ids[:, j * SPG:(j + 1) * SPG]
        m = jnp.max(blk, axis=1)
        ch = jnp.min(jnp.where(blk == m[:, None], io, SPG), axis=1)
        sid = jnp.max(jnp.where(io == ch[:, None], idb, -1), axis=1)
        cols.append(sid[:, None])
        maxes.append(m[:, None])
    syn_ref[...] = jnp.concatenate(cols, axis=1)
    sc_ref[...] = jnp.concatenate(maxes, axis=1)


def _epilogue(syn_logits, g_s2, syn_ids, sel_logit):
    return pl.pallas_call(
        _epilogue_body,
        out_shape=[
            jax.ShapeDtypeStruct((B, NRG), jnp.int32),
            jax.ShapeDtypeStruct((B, 1 + NRG), jnp.float32),
        ],
    )(syn_logits, g_s2, syn_ids, sel_logit)


def kernel(queries, reaction_keys, synthon_keys, rgroup2synthons, noise_r, noise_s):
    g_s = -jnp.log(-jnp.log(noise_s + EPS) + EPS)

    reactions, sel_logit = _reaction_phase(queries, reaction_keys, noise_r)

    syn_logits, syn_ids = _synthon_phase(
        reactions, queries,
        rgroup2synthons.reshape(R, NRG * SPG), synthon_keys)

    synthons, scores = _epilogue(
        syn_logits, g_s.reshape(B, NCAND), syn_ids, sel_logit)
    return reactions, synthons, scores


# speedup vs baseline: 1.0562x; 1.0005x over previous
import functools

import jax
import jax.numpy as jnp
from jax import lax
from jax.experimental import pallas as pl
from jax.experimental.pallas import tpu as pltpu
from jax.experimental.pallas import tpu_sc as plsc

B = 512
D = 512
R = 16384
NRG = 3
SPG = 32
S = 100000
G = R * NRG
NCAND = NRG * SPG
EPS = 1e-9

R_BLK = 2048
NEG_INF = float('-inf')

NWORKER = 32
QPW = B // NWORKER


def _reaction_body(q_ref, k_ref, gr_ref, react_ref, sel_ref,
                   max_sc, idx_sc, sel_sc):
    i = pl.program_id(0)

    @pl.when(i == 0)
    def _init():
        max_sc[...] = jnp.full((B,), NEG_INF, jnp.float32)
        idx_sc[...] = jnp.zeros((B,), jnp.int32)
        sel_sc[...] = jnp.zeros((B,), jnp.float32)

    logits = lax.dot_general(
        q_ref[...], k_ref[...],
        dimension_numbers=(((1,), (1,)), ((), ())),
        preferred_element_type=jnp.float32)
    g_r = -jnp.log(-jnp.log(gr_ref[...] + EPS) + EPS)
    pert = logits + g_r
    pmax = jnp.max(pert, axis=1)
    iota = lax.broadcasted_iota(jnp.int32, (B, R_BLK), 1)
    loc = jnp.min(jnp.where(pert == pmax[:, None], iota, R_BLK), axis=1)
    sel = jnp.max(jnp.where(iota == loc[:, None], logits, NEG_INF), axis=1)

    better = pmax > max_sc[...]
    max_sc[...] = jnp.where(better, pmax, max_sc[...])
    idx_sc[...] = jnp.where(better, loc + i * R_BLK, idx_sc[...])
    sel_sc[...] = jnp.where(better, sel, sel_sc[...])

    @pl.when(i == pl.num_programs(0) - 1)
    def _done():
        react_ref[...] = idx_sc[...]
        sel_ref[...] = sel_sc[...]


def _reaction_phase(queries, reaction_keys, g_r):
    return pl.pallas_call(
        _reaction_body,
        grid=(R // R_BLK,),
        in_specs=[
            pl.BlockSpec((B, D), lambda i: (0, 0)),
            pl.BlockSpec((R_BLK, D), lambda i: (i, 0)),
            pl.BlockSpec((B, R_BLK), lambda i: (0, i)),
        ],
        out_specs=[
            pl.BlockSpec((B,), lambda i: (0,)),
            pl.BlockSpec((B,), lambda i: (0,)),
        ],
        out_shape=[
            jax.ShapeDtypeStruct((B,), jnp.int32),
            jax.ShapeDtypeStruct((B,), jnp.float32),
        ],
        scratch_shapes=[
            pltpu.VMEM((B,), jnp.float32),
            pltpu.VMEM((B,), jnp.int32),
            pltpu.VMEM((B,), jnp.float32),
        ],
        compiler_params=pltpu.CompilerParams(
            dimension_semantics=("arbitrary",)),
    )(queries, reaction_keys, g_r)


def _rne_bf16(v):
    x = plsc.bitcast(v, jnp.int32)
    lsb = (x >> 16) & 1
    r = (x + (0x7FFF + lsb)) & jnp.int32(-65536)
    return plsc.bitcast(r, jnp.float32)


def _synthon_sc_body(react_hbm, q_hbm, r2s_hbm, skeys_hbm,
                     out_logits_hbm, out_ids_hbm,
                     react_v, flat_v, q_v, rows2,
                     part_t, stage_v, sem0, sem1):
    wid = lax.axis_index("s") * 2 + lax.axis_index("c")
    base = wid * QPW

    pltpu.sync_copy(react_hbm.at[pl.ds(base, QPW)], react_v)
    pltpu.sync_copy(q_hbm.at[pl.ds(base, QPW)], q_v)

    iota = lax.iota(jnp.int32, 16)
    r_vec = react_v[...]
    for b in range(QPW):
        rc = r_vec[b]
        pltpu.async_copy(r2s_hbm.at[pl.ds(rc, 1)], flat_v.at[pl.ds(b, 1)],
                         sem1)
    for b in range(QPW):
        pltpu.make_async_copy(r2s_hbm.at[pl.ds(0, 1)],
                              flat_v.at[pl.ds(b, 1)], sem1).wait()
    pltpu.async_copy(skeys_hbm.at[flat_v.at[0]], rows2.at[0], sem0)
    pltpu.sync_copy(flat_v, out_ids_hbm.at[pl.ds(base, QPW)])

    for b in range(QPW):
        p = b % 2
        cur_sem = sem0 if p == 0 else sem1
        if b < QPW - 1:
            nxt_sem = sem1 if p == 0 else sem0
            pltpu.async_copy(skeys_hbm.at[flat_v.at[b + 1]],
                             rows2.at[1 - p], nxt_sem)
        pltpu.make_async_copy(skeys_hbm.at[flat_v.at[b]],
                              rows2.at[p], cur_sem).wait()

        for c in range(D // 16):
            q_v[b, pl.ds(c * 16, 16)] = _rne_bf16(q_v[b, pl.ds(c * 16, 16)])

        def _cand_body(ci, _c, p=p, b=b):
            accs = [jnp.zeros((16,), jnp.float32) for _ in range(4)]
            for c in range(D // 16):
                k = _rne_bf16(rows2[p, ci, pl.ds(c * 16, 16)])
                accs[c % 4] = accs[c % 4] + k * q_v[b, pl.ds(c * 16, 16)]
            acc = (accs[0] + accs[1]) + (accs[2] + accs[3])
            plsc.store_scatter(part_t, [iota, jnp.full((16,), ci, jnp.int32)],
                               acc)
            return _c
        lax.fori_loop(0, NCAND, _cand_body, 0)

        def _red_body(g, _r2, b=b):
            tot = jnp.zeros((16,), jnp.float32)
            for k in range(16):
                tot = tot + part_t[k, pl.ds(g * 16, 16)]
            stage_v[b, pl.ds(g * 16, 16)] = tot
            return _r2
        lax.fori_loop(0, NCAND // 16, _red_body, 0)

    pltpu.sync_copy(stage_v, out_logits_hbm.at[pl.ds(base, QPW)])


def _synthon_phase(reactions, queries_rnd, rgroup2synthons, synthon_keys):
    mesh = plsc.VectorSubcoreMesh(core_axis_name="c", subcore_axis_name="s")
    fn = pl.kernel(
        _synthon_sc_body,
        out_type=[
            jax.ShapeDtypeStruct((B, NCAND), jnp.float32),
            jax.ShapeDtypeStruct((B, NCAND), jnp.int32),
        ],
        mesh=mesh,
        scratch_types=[
            pltpu.VMEM((QPW,), jnp.int32),
            pltpu.VMEM((QPW, NCAND), jnp.int32),
            pltpu.VMEM((QPW, D), jnp.float32),
            pltpu.VMEM((2, NCAND, D), jnp.float32),
            pltpu.VMEM((16, NCAND), jnp.float32),
            pltpu.VMEM((QPW, NCAND), jnp.float32),
            pltpu.SemaphoreType.DMA,
            pltpu.SemaphoreType.DMA,
        ],
        compiler_params=pltpu.CompilerParams(needs_layout_passes=False),
    )
    return fn(reactions, queries_rnd, rgroup2synthons, synthon_keys)


def _epilogue_body(lg_ref, gs_ref, ids_ref, sel_ref, syn_ref, sc_ref):
    pert = lg_ref[...] + gs_ref[...]
    ids = ids_ref[...]
    cols = []
    maxes = [sel_ref[...][:, None]]
    io = lax.broadcasted_iota(jnp.int32, (B, SPG), 1)
    for j in range(NRG):
        blk = pert[:, j * SPG:(j + 1) * SPG]
        idb = ids[:, j * SPG:(j + 1) * SPG]
        m = jnp.max(blk, axis=1)
        ch = jnp.min(jnp.where(blk == m[:, None], io, SPG), axis=1)
        sid = jnp.max(jnp.where(io == ch[:, None], idb, -1), axis=1)
        cols.append(sid[:, None])
        maxes.append(m[:, None])
    syn_ref[...] = jnp.concatenate(cols, axis=1)
    sc_ref[...] = jnp.concatenate(maxes, axis=1)


def _epilogue(syn_logits, g_s2, syn_ids, sel_logit):
    return pl.pallas_call(
        _epilogue_body,
        out_shape=[
            jax.ShapeDtypeStruct((B, NRG), jnp.int32),
            jax.ShapeDtypeStruct((B, 1 + NRG), jnp.float32),
        ],
    )(syn_logits, g_s2, syn_ids, sel_logit)


def kernel(queries, reaction_keys, synthon_keys, rgroup2synthons, noise_r, noise_s):
    g_s = -jnp.log(-jnp.log(noise_s + EPS) + EPS)

    reactions, sel_logit = _reaction_phase(queries, reaction_keys, noise_r)

    syn_logits, syn_ids = _synthon_phase(
        reactions, queries,
        rgroup2synthons.reshape(R, NRG * SPG), synthon_keys)

    synthons, scores = _epilogue(
        syn_logits, g_s.reshape(B, NCAND), syn_ids, sel_logit)
    return reactions, synthons, scores
